# baseline (speedup 1.0000x reference)
import functools

import jax
import jax.numpy as jnp
from jax import lax
from jax.experimental import pallas as pl
from jax.experimental.pallas import tpu as pltpu
from jax.experimental.pallas import tpu_sc as plsc

N = 50000
X = 256
H = 256
K_ITERS = 6
HIDDEN = 50
NUM_CLASSES = 5

RB = 512
NP = ((N + RB - 1) // RB) * RB

T = 128
R = 2048
CH = 26
CPC = CH // 2
TRR = R // 16
TRASH = R


TE = T // 2
UTR = 2 * TRR
TRASHU = 2 * R


def _tree_pass(vals, gidx, dstl, out, cs_vm, cs_base, s, t0u,
               gi0, di0, rows0, gi1, di1, rows1, zbuf, acc, sem0, sem1, c):
    def chunk_body(j, carry):
        k = 2 * j + lax.bitwise_xor(c, lax.bitwise_and(j, 1))
        st = cs_vm[pl.ds(cs_base + k, 16)][0]
        en = cs_vm[pl.ds(cs_base + k + 1, 16)][0]
        cnt = en - st
        per = (cnt + 15) // 16
        s_t = st + s * per
        e_t = jnp.minimum(s_t + per, en)
        b = lax.bitwise_and(s_t, -4)
        nsteps = jnp.where(e_t > s_t, (e_t - b + TE - 1) // TE, 0)

        pltpu.sync_copy(zbuf, acc.at[pl.ds(t0u, UTR)])
        plsc.subcore_barrier()

        def load(i2, gi, di):
            qoff = pl.multiple_of(2 * b + i2 * T, 8)
            pltpu.sync_copy(gidx.at[pl.ds(qoff, T)], gi)
            pltpu.sync_copy(dstl.at[pl.ds(qoff, T)], di)

            def maskv(v, cc):
                q = qoff + v * 16 + lax.broadcasted_iota(jnp.int32, (16,), 0)
                e_idx = lax.shift_right_logical(q, 1)
                dv = di[pl.ds(v * 16, 16)]
                ok = (e_idx >= s_t) & (e_idx < e_t)
                di[pl.ds(v * 16, 16)] = jnp.where(ok, dv, TRASHU)
                return cc

            lax.fori_loop(0, T // 16, maskv, 0)

        def fire(gi, rows, sem):
            pltpu.make_async_copy(vals.at[gi], rows, sem).start()

        def drain(gi, di, rows, sem):
            pltpu.make_async_copy(vals.at[gi], rows, sem).wait()
            pltpu.async_copy(rows, acc.at[di], sem, add=True).wait()

        @pl.when(nsteps > 0)
        def _prologue():
            load(0, gi0, di0)
            fire(gi0, rows0, sem0)

        def pair(p, carry2):
            i_odd = 2 * p + 1

            @pl.when(i_odd < nsteps)
            def _fire_odd():
                load(i_odd, gi1, di1)
                fire(gi1, rows1, sem1)

            drain(gi0, di0, rows0, sem0)

            @pl.when(i_odd < nsteps)
            def _odd_half():
                @pl.when(i_odd + 1 < nsteps)
                def _fire_next_even():
                    load(i_odd + 1, gi0, di0)
                    fire(gi0, rows0, sem0)

                drain(gi1, di1, rows1, sem1)

            return carry2

        lax.fori_loop(0, (nsteps + 1) // 2, pair, 0)
        plsc.subcore_barrier()
        pltpu.sync_copy(acc.at[pl.ds(t0u, UTR)],
                        out.at[pl.ds(2 * k * R + t0u, UTR)])
        return carry

    lax.fori_loop(0, CPC, chunk_body, 0)


def _sc_scatter_body(vals_a, gidx_a, dstl_a, vals_b, gidx_b, dstl_b, cs, zrows,
                     out_a, out_b,
                     gi0, di0, rows0, gi1, di1, rows1,
                     zbuf, cs_vm, acc, sem0, sem1):
    c = lax.axis_index("c")
    s = lax.axis_index("s")
    pltpu.sync_copy(cs, cs_vm)
    pltpu.sync_copy(zrows, zbuf)
    t0u = s * UTR
    _tree_pass(vals_a, gidx_a, dstl_a, out_a, cs_vm, 0, s, t0u,
               gi0, di0, rows0, gi1, di1, rows1, zbuf, acc, sem0, sem1, c)
    _tree_pass(vals_b, gidx_b, dstl_b, out_b, cs_vm, 32, s, t0u,
               gi0, di0, rows0, gi1, di1, rows1, zbuf, acc, sem0, sem1, c)


_sc_scatter = functools.partial(
    pl.kernel,
    mesh=plsc.VectorSubcoreMesh(core_axis_name="c", subcore_axis_name="s"),
    out_type=[jax.ShapeDtypeStruct((2 * CH * R, 128), jnp.float32)] * 2,
    scratch_types=[
        pltpu.VMEM((T,), jnp.int32),
        pltpu.VMEM((T,), jnp.int32),
        pltpu.VMEM((T, 128), jnp.float32),
        pltpu.VMEM((T,), jnp.int32),
        pltpu.VMEM((T,), jnp.int32),
        pltpu.VMEM((T, 128), jnp.float32),
        pltpu.VMEM((UTR, 128), jnp.float32),
        pltpu.VMEM((64,), jnp.int32),
        pltpu.VMEM_SHARED((2 * R + 16, 128), jnp.float32),
        pltpu.SemaphoreType.DMA,
        pltpu.SemaphoreType.DMA,
    ],
)(_sc_scatter_body)


def _scatter_add2(arr_a, ed_a, arr_b, ed_b, cs2, zrows):
    va = jnp.reshape(arr_a, (2 * NP, 128))
    vb = jnp.reshape(arr_b, (2 * NP, 128))
    oa, ob = _sc_scatter(va, ed_a[0], ed_a[1], vb, ed_b[0], ed_b[1], cs2, zrows)
    return jnp.reshape(oa, (CH * R, H)), jnp.reshape(ob, (CH * R, H))


def _build_edge_data(parent):
    par = parent[1:N].astype(jnp.int32)
    order = jnp.argsort(par).astype(jnp.int32)
    ps = par[order]
    child = order + 1
    child2 = jnp.stack([2 * child, 2 * child + 1], axis=1).reshape(-1)
    du = 2 * (ps % R)
    dst2 = jnp.stack([du, du + 1], axis=1).reshape(-1)
    gidx = jnp.zeros((2 * NP,), jnp.int32).at[:2 * (N - 1)].set(child2)
    dstl = jnp.full((2 * NP,), TRASHU, jnp.int32).at[:2 * (N - 1)].set(dst2)
    bounds = jnp.arange(CH + 1, dtype=jnp.int32) * R
    cs = jnp.searchsorted(ps, bounds).astype(jnp.int32)
    return gidx, dstl, cs


def _embed_mm_body(x_ref, wiou_ref, biou_ref, wf_ref, bf_ref, xw_ref, xwf_ref):
    x = x_ref[...]
    xw_ref[...] = jnp.dot(x, wiou_ref[...],
                          preferred_element_type=jnp.float32) + biou_ref[...]
    xwf_ref[...] = jnp.dot(x, wf_ref[...],
                           preferred_element_type=jnp.float32) + bf_ref[...]


def _embed_mm(x, W_iou, b_iou, W_f, b_f):
    grid = (NP // RB,)
    return pl.pallas_call(
        _embed_mm_body,
        grid=grid,
        in_specs=[
            pl.BlockSpec((RB, X), lambda i: (i, 0)),
            pl.BlockSpec((X, 3 * H), lambda i: (0, 0)),
            pl.BlockSpec((1, 3 * H), lambda i: (0, 0)),
            pl.BlockSpec((X, H), lambda i: (0, 0)),
            pl.BlockSpec((1, H), lambda i: (0, 0)),
        ],
        out_specs=[
            pl.BlockSpec((RB, 3 * H), lambda i: (i, 0)),
            pl.BlockSpec((RB, H), lambda i: (i, 0)),
        ],
        out_shape=[
            jax.ShapeDtypeStruct((NP, 3 * H), jnp.float32),
            jax.ShapeDtypeStruct((NP, H), jnp.float32),
        ],
    )(x, W_iou, b_iou.reshape(1, -1), W_f, b_f.reshape(1, -1))


def _iter1_body(xw_ref, c_ref, h_ref):
    xw = xw_ref[...]
    i_g = jax.nn.sigmoid(xw[:, :H])
    o_g = jax.nn.sigmoid(xw[:, H:2 * H])
    u_g = jnp.tanh(xw[:, 2 * H:])
    c = i_g * u_g
    c_ref[...] = c
    h_ref[...] = o_g * jnp.tanh(c)


def _iter1(xw):
    grid = (NP // RB,)
    return pl.pallas_call(
        _iter1_body,
        grid=grid,
        in_specs=[pl.BlockSpec((RB, 3 * H), lambda i: (i, 0))],
        out_specs=[pl.BlockSpec((RB, H), lambda i: (i, 0)),
                   pl.BlockSpec((RB, H), lambda i: (i, 0))],
        out_shape=[jax.ShapeDtypeStruct((NP, H), jnp.float32),
                   jax.ShapeDtypeStruct((NP, H), jnp.float32)],
    )(xw)


def _step_a_body(xw_ref, fp_ref, h_ref, hs_ref, c_ref, uiou_ref, uf_ref,
                 iu_ref, o_ref, fcc_ref):
    iou = xw_ref[...] + jnp.dot(hs_ref[...], uiou_ref[...],
                                preferred_element_type=jnp.float32)
    i_g = jax.nn.sigmoid(iou[:, :H])
    o_g = jax.nn.sigmoid(iou[:, H:2 * H])
    u_g = jnp.tanh(iou[:, 2 * H:])
    iu_ref[...] = i_g * u_g
    o_ref[...] = o_g
    fe = jax.nn.sigmoid(fp_ref[...] + jnp.dot(h_ref[...], uf_ref[...],
                                              preferred_element_type=jnp.float32))
    fcc_ref[...] = fe * c_ref[...]


def _step_a(xw, fp, h, h_sum, c, U_iou, U_f):
    grid = (NP // RB,)
    return pl.pallas_call(
        _step_a_body,
        grid=grid,
        in_specs=[
            pl.BlockSpec((RB, 3 * H), lambda i: (i, 0)),
            pl.BlockSpec((RB, H), lambda i: (i, 0)),
            pl.BlockSpec((RB, H), lambda i: (i, 0)),
            pl.BlockSpec((RB, H), lambda i: (i, 0)),
            pl.BlockSpec((RB, H), lambda i: (i, 0)),
            pl.BlockSpec((H, 3 * H), lambda i: (0, 0)),
            pl.BlockSpec((H, H), lambda i: (0, 0)),
        ],
        out_specs=[pl.BlockSpec((RB, H), lambda i: (i, 0))] * 3,
        out_shape=[jax.ShapeDtypeStruct((NP, H), jnp.float32)] * 3,
    )(xw, fp, h, h_sum, c, U_iou, U_f)


def _step_b_body(iu_ref, o_ref, fc_ref, c_ref, h_ref):
    c = iu_ref[...] + fc_ref[...]
    c_ref[...] = c
    h_ref[...] = o_ref[...] * jnp.tanh(c)


def _step_b(iu, o, fc):
    grid = (NP // RB,)
    return pl.pallas_call(
        _step_b_body,
        grid=grid,
        in_specs=[pl.BlockSpec((RB, H), lambda i: (i, 0))] * 3,
        out_specs=[pl.BlockSpec((RB, H), lambda i: (i, 0))] * 2,
        out_shape=[jax.ShapeDtypeStruct((NP, H), jnp.float32)] * 2,
    )(iu, o, fc)


def _head_body(ha_ref, hb_ref, whw_ref, whb_ref, wpw_ref, wpb_ref, r_ref,
               out_ref, pred_ref):
    lvec = ha_ref[...]
    rvec = hb_ref[...]
    vec = jnp.concatenate([lvec * rvec, jnp.abs(lvec - rvec)], axis=1)
    hid = jax.nn.sigmoid(jnp.dot(vec, whw_ref[...],
                                 preferred_element_type=jnp.float32) + whb_ref[...])
    hcol = jax.lax.broadcasted_iota(jnp.int32, hid.shape, 1)
    hid = jnp.where(hcol < HIDDEN, hid, 0.0)
    logits = jnp.dot(hid, wpw_ref[...],
                     preferred_element_type=jnp.float32) + wpb_ref[...]
    col = jax.lax.broadcasted_iota(jnp.int32, logits.shape, 1)
    valid = col < NUM_CLASSES
    masked = jnp.where(valid, logits, -jnp.inf)
    m = jnp.max(masked, axis=1, keepdims=True)
    e = jnp.where(valid, jnp.exp(logits - m), 0.0)
    lse = m + jnp.log(jnp.sum(e, axis=1, keepdims=True))
    lsm = logits - lse
    out_ref[...] = lsm
    p = jnp.sum(jnp.where(valid, jnp.exp(lsm), 0.0) * r_ref[...], axis=1,
                keepdims=True)
    pred_ref[...] = jnp.broadcast_to(p, pred_ref.shape)


def _head(ha8, hb8, wh_W, wh_b, wp_W, wp_b, r):
    whw = jnp.zeros((2 * H, 64), jnp.float32).at[:, :HIDDEN].set(wh_W)
    whb = jnp.zeros((1, 64), jnp.float32).at[0, :HIDDEN].set(wh_b)
    wpw = jnp.zeros((64, 128), jnp.float32).at[:HIDDEN, :NUM_CLASSES].set(wp_W)
    wpb = jnp.zeros((1, 128), jnp.float32).at[0, :NUM_CLASSES].set(wp_b)
    rp = jnp.zeros((1, 128), jnp.float32).at[0, :NUM_CLASSES].set(r)
    out, pred = pl.pallas_call(
        _head_body,
        in_specs=[pl.BlockSpec((8, H), lambda: (0, 0)),
                  pl.BlockSpec((8, H), lambda: (0, 0)),
                  pl.BlockSpec((2 * H, 64), lambda: (0, 0)),
                  pl.BlockSpec((1, 64), lambda: (0, 0)),
                  pl.BlockSpec((64, 128), lambda: (0, 0)),
                  pl.BlockSpec((1, 128), lambda: (0, 0)),
                  pl.BlockSpec((1, 128), lambda: (0, 0))],
        out_specs=[pl.BlockSpec((8, 128), lambda: (0, 0)),
                   pl.BlockSpec((8, 128), lambda: (0, 0))],
        out_shape=[jax.ShapeDtypeStruct((8, 128), jnp.float32),
                   jax.ShapeDtypeStruct((8, 128), jnp.float32)],
    )(ha8, hb8, whw, whb, wpw, wpb, rp)
    return out[0:1, :NUM_CLASSES], pred[0:1, 0]


def _tree_setup(x_ids, parent, emb, W_iou, b_iou, W_f, b_f):
    ids = jnp.zeros((NP,), x_ids.dtype).at[:N].set(x_ids)
    parp = jnp.zeros((NP,), parent.dtype).at[:N].set(parent)
    x = jnp.take(emb, ids, axis=0)
    xw, xwf = _embed_mm(x, W_iou, b_iou, W_f, b_f)
    fp = jnp.take(xwf, parp, axis=0)
    gidx, dstl, cs = _build_edge_data(parent)
    c, h = _iter1(xw)
    return xw, fp, (gidx, dstl), cs, c, h


def kernel(x_ids_a, parent_a, x_ids_b, parent_b, emb, W_iou, U_iou, b_iou,
           W_f, U_f, b_f, wh_W, wh_b, wp_W, wp_b, r):
    xw_a, fp_a, ed_a, cs_a, c_a, h_a = _tree_setup(
        x_ids_a, parent_a, emb, W_iou, b_iou, W_f, b_f)
    xw_b, fp_b, ed_b, cs_b, c_b, h_b = _tree_setup(
        x_ids_b, parent_b, emb, W_iou, b_iou, W_f, b_f)
    cs2 = jnp.zeros((64,), jnp.int32).at[:CH + 1].set(cs_a).at[32:32 + CH + 1].set(cs_b)
    zrows = jnp.zeros((UTR, 128), jnp.float32)
    for _ in range(K_ITERS - 1):
        hs_a, hs_b = _scatter_add2(h_a, ed_a, h_b, ed_b, cs2, zrows)
        iu_a, o_a, fcc_a = _step_a(xw_a, fp_a, h_a, hs_a, c_a, U_iou, U_f)
        iu_b, o_b, fcc_b = _step_a(xw_b, fp_b, h_b, hs_b, c_b, U_iou, U_f)
        fc_a, fc_b = _scatter_add2(fcc_a, ed_a, fcc_b, ed_b, cs2, zrows)
        c_a, h_a = _step_b(iu_a, o_a, fc_a)
        c_b, h_b = _step_b(iu_b, o_b, fc_b)
    return _head(h_a[0:8], h_b[0:8], wh_W, wh_b, wp_W, wp_b, r)


# speedup vs baseline: 1.3058x; 1.3058x over previous
import functools

import jax
import jax.numpy as jnp
from jax import lax
from jax.experimental import pallas as pl
from jax.experimental.pallas import tpu as pltpu
from jax.experimental.pallas import tpu_sc as plsc

N = 50000
X = 256
H = 256
K_ITERS = 6
HIDDEN = 50
NUM_CLASSES = 5

RB = 512
NP = ((N + RB - 1) // RB) * RB

T = 128
R = 2048
CH = 26
CPC = CH // 2
TRR = R // 16
TRASH = R


TE = T // 2
UTR = 2 * TRR
TRASHU = 2 * R


def _tree_pass(vals, gidx, dstl, out, cs_vm, cs_base, s, t0u,
               gi0, di0, rows0, gi1, di1, rows1, zbuf, acc, sem0, sem1, c):
    def chunk_body(j, carry):
        k = 2 * j + lax.bitwise_xor(c, lax.bitwise_and(j, 1))
        st = cs_vm[pl.ds(cs_base + k, 16)][0]
        en = cs_vm[pl.ds(cs_base + k + 1, 16)][0]
        cnt = en - st
        per = (cnt + 15) // 16
        s_t = st + s * per
        e_t = jnp.minimum(s_t + per, en)
        b = lax.bitwise_and(s_t, -4)
        nsteps = jnp.where(e_t > s_t, (e_t - b + TE - 1) // TE, 0)

        pltpu.sync_copy(zbuf, acc.at[pl.ds(t0u, UTR)])
        plsc.subcore_barrier()

        def load(i2, gi, di):
            qoff = pl.multiple_of(2 * b + i2 * T, 8)
            pltpu.sync_copy(gidx.at[pl.ds(qoff, T)], gi)
            pltpu.sync_copy(dstl.at[pl.ds(qoff, T)], di)

            def maskv(v, cc):
                q = qoff + v * 16 + lax.broadcasted_iota(jnp.int32, (16,), 0)
                e_idx = lax.shift_right_logical(q, 1)
                dv = di[pl.ds(v * 16, 16)]
                ok = (e_idx >= s_t) & (e_idx < e_t)
                di[pl.ds(v * 16, 16)] = jnp.where(ok, dv, TRASHU)
                return cc

            lax.fori_loop(0, T // 16, maskv, 0)

        def fire(gi, rows, sem):
            pltpu.make_async_copy(vals.at[gi], rows, sem).start()

        def drain(gi, di, rows, sem):
            pltpu.make_async_copy(vals.at[gi], rows, sem).wait()
            pltpu.async_copy(rows, acc.at[di], sem, add=True).wait()

        @pl.when(nsteps > 0)
        def _prologue():
            load(0, gi0, di0)
            fire(gi0, rows0, sem0)

        def pair(p, carry2):
            i_odd = 2 * p + 1

            @pl.when(i_odd < nsteps)
            def _fire_odd():
                load(i_odd, gi1, di1)
                fire(gi1, rows1, sem1)

            drain(gi0, di0, rows0, sem0)

            @pl.when(i_odd < nsteps)
            def _odd_half():
                @pl.when(i_odd + 1 < nsteps)
                def _fire_next_even():
                    load(i_odd + 1, gi0, di0)
                    fire(gi0, rows0, sem0)

                drain(gi1, di1, rows1, sem1)

            return carry2

        lax.fori_loop(0, (nsteps + 1) // 2, pair, 0)
        plsc.subcore_barrier()
        pltpu.sync_copy(acc.at[pl.ds(t0u, UTR)],
                        out.at[pl.ds(2 * k * R + t0u, UTR)])
        return carry

    lax.fori_loop(0, CPC, chunk_body, 0)


def _sc_scatter_body(vals, gidx, dstl, cs, zrows, out,
                     gi0, di0, rows0, gi1, di1, rows1,
                     zbuf, cs_vm, acc, sem0, sem1):
    c = lax.axis_index("c")
    s = lax.axis_index("s")
    pltpu.sync_copy(cs, cs_vm)
    pltpu.sync_copy(zrows, zbuf)
    t0u = s * UTR
    _tree_pass(vals, gidx, dstl, out, cs_vm, 0, s, t0u,
               gi0, di0, rows0, gi1, di1, rows1, zbuf, acc, sem0, sem1, c)


_sc_scatter = functools.partial(
    pl.kernel,
    mesh=plsc.VectorSubcoreMesh(core_axis_name="c", subcore_axis_name="s"),
    out_type=jax.ShapeDtypeStruct((2 * CH * R, 128), jnp.float32),
    scratch_types=[
        pltpu.VMEM((T,), jnp.int32),
        pltpu.VMEM((T,), jnp.int32),
        pltpu.VMEM((T, 128), jnp.float32),
        pltpu.VMEM((T,), jnp.int32),
        pltpu.VMEM((T,), jnp.int32),
        pltpu.VMEM((T, 128), jnp.float32),
        pltpu.VMEM((UTR, 128), jnp.float32),
        pltpu.VMEM((64,), jnp.int32),
        pltpu.VMEM_SHARED((2 * R + 16, 128), jnp.float32),
        pltpu.SemaphoreType.DMA,
        pltpu.SemaphoreType.DMA,
    ],
)(_sc_scatter_body)


def _scatter_add(arr, ed, cs2, zrows):
    v = jnp.reshape(arr, (2 * NP, 128))
    o = _sc_scatter(v, ed[0], ed[1], cs2, zrows)
    return jnp.reshape(o, (CH * R, H))


def _build_edge_data(parent):
    par = parent[1:N].astype(jnp.int32)
    order = jnp.argsort(par).astype(jnp.int32)
    ps = par[order]
    child = order + 1
    child2 = jnp.stack([2 * child, 2 * child + 1], axis=1).reshape(-1)
    du = 2 * (ps % R)
    dst2 = jnp.stack([du, du + 1], axis=1).reshape(-1)
    gidx = jnp.zeros((2 * NP,), jnp.int32).at[:2 * (N - 1)].set(child2)
    dstl = jnp.full((2 * NP,), TRASHU, jnp.int32).at[:2 * (N - 1)].set(dst2)
    bounds = jnp.arange(CH + 1, dtype=jnp.int32) * R
    cs = jnp.searchsorted(ps, bounds).astype(jnp.int32)
    return gidx, dstl, cs


def _embed_mm_body(x_ref, wiou_ref, biou_ref, wf_ref, bf_ref, xw_ref, xwf_ref):
    x = x_ref[...]
    xw_ref[...] = jnp.dot(x, wiou_ref[...],
                          preferred_element_type=jnp.float32) + biou_ref[...]
    xwf_ref[...] = jnp.dot(x, wf_ref[...],
                           preferred_element_type=jnp.float32) + bf_ref[...]


def _embed_mm(x, W_iou, b_iou, W_f, b_f):
    grid = (NP // RB,)
    return pl.pallas_call(
        _embed_mm_body,
        grid=grid,
        in_specs=[
            pl.BlockSpec((RB, X), lambda i: (i, 0)),
            pl.BlockSpec((X, 3 * H), lambda i: (0, 0)),
            pl.BlockSpec((1, 3 * H), lambda i: (0, 0)),
            pl.BlockSpec((X, H), lambda i: (0, 0)),
            pl.BlockSpec((1, H), lambda i: (0, 0)),
        ],
        out_specs=[
            pl.BlockSpec((RB, 3 * H), lambda i: (i, 0)),
            pl.BlockSpec((RB, H), lambda i: (i, 0)),
        ],
        out_shape=[
            jax.ShapeDtypeStruct((NP, 3 * H), jnp.float32),
            jax.ShapeDtypeStruct((NP, H), jnp.float32),
        ],
    )(x, W_iou, b_iou.reshape(1, -1), W_f, b_f.reshape(1, -1))


def _iter1_body(xw_ref, c_ref, h_ref):
    xw = xw_ref[...]
    i_g = jax.nn.sigmoid(xw[:, :H])
    o_g = jax.nn.sigmoid(xw[:, H:2 * H])
    u_g = jnp.tanh(xw[:, 2 * H:])
    c = i_g * u_g
    c_ref[...] = c
    h_ref[...] = o_g * jnp.tanh(c)


def _iter1(xw):
    grid = (NP // RB,)
    return pl.pallas_call(
        _iter1_body,
        grid=grid,
        in_specs=[pl.BlockSpec((RB, 3 * H), lambda i: (i, 0))],
        out_specs=[pl.BlockSpec((RB, H), lambda i: (i, 0)),
                   pl.BlockSpec((RB, H), lambda i: (i, 0))],
        out_shape=[jax.ShapeDtypeStruct((NP, H), jnp.float32),
                   jax.ShapeDtypeStruct((NP, H), jnp.float32)],
    )(xw)


def _step_a_body(xw_ref, fp_ref, h_ref, hs_ref, c_ref, uiou_ref, uf_ref,
                 iu_ref, o_ref, fcc_ref):
    iou = xw_ref[...] + jnp.dot(hs_ref[...], uiou_ref[...],
                                preferred_element_type=jnp.float32)
    i_g = jax.nn.sigmoid(iou[:, :H])
    o_g = jax.nn.sigmoid(iou[:, H:2 * H])
    u_g = jnp.tanh(iou[:, 2 * H:])
    iu_ref[...] = i_g * u_g
    o_ref[...] = o_g
    fe = jax.nn.sigmoid(fp_ref[...] + jnp.dot(h_ref[...], uf_ref[...],
                                              preferred_element_type=jnp.float32))
    fcc_ref[...] = fe * c_ref[...]


def _step_a(xw, fp, h, h_sum, c, U_iou, U_f):
    grid = (NP // RB,)
    return pl.pallas_call(
        _step_a_body,
        grid=grid,
        in_specs=[
            pl.BlockSpec((RB, 3 * H), lambda i: (i, 0)),
            pl.BlockSpec((RB, H), lambda i: (i, 0)),
            pl.BlockSpec((RB, H), lambda i: (i, 0)),
            pl.BlockSpec((RB, H), lambda i: (i, 0)),
            pl.BlockSpec((RB, H), lambda i: (i, 0)),
            pl.BlockSpec((H, 3 * H), lambda i: (0, 0)),
            pl.BlockSpec((H, H), lambda i: (0, 0)),
        ],
        out_specs=[pl.BlockSpec((RB, H), lambda i: (i, 0))] * 3,
        out_shape=[jax.ShapeDtypeStruct((NP, H), jnp.float32)] * 3,
    )(xw, fp, h, h_sum, c, U_iou, U_f)


def _step_b_body(iu_ref, o_ref, fc_ref, c_ref, h_ref):
    c = iu_ref[...] + fc_ref[...]
    c_ref[...] = c
    h_ref[...] = o_ref[...] * jnp.tanh(c)


def _step_b(iu, o, fc):
    grid = (NP // RB,)
    return pl.pallas_call(
        _step_b_body,
        grid=grid,
        in_specs=[pl.BlockSpec((RB, H), lambda i: (i, 0))] * 3,
        out_specs=[pl.BlockSpec((RB, H), lambda i: (i, 0))] * 2,
        out_shape=[jax.ShapeDtypeStruct((NP, H), jnp.float32)] * 2,
    )(iu, o, fc)


def _head_body(ha_ref, hb_ref, whw_ref, whb_ref, wpw_ref, wpb_ref, r_ref,
               out_ref, pred_ref):
    lvec = ha_ref[...]
    rvec = hb_ref[...]
    vec = jnp.concatenate([lvec * rvec, jnp.abs(lvec - rvec)], axis=1)
    hid = jax.nn.sigmoid(jnp.dot(vec, whw_ref[...],
                                 preferred_element_type=jnp.float32) + whb_ref[...])
    hcol = jax.lax.broadcasted_iota(jnp.int32, hid.shape, 1)
    hid = jnp.where(hcol < HIDDEN, hid, 0.0)
    logits = jnp.dot(hid, wpw_ref[...],
                     preferred_element_type=jnp.float32) + wpb_ref[...]
    col = jax.lax.broadcasted_iota(jnp.int32, logits.shape, 1)
    valid = col < NUM_CLASSES
    masked = jnp.where(valid, logits, -jnp.inf)
    m = jnp.max(masked, axis=1, keepdims=True)
    e = jnp.where(valid, jnp.exp(logits - m), 0.0)
    lse = m + jnp.log(jnp.sum(e, axis=1, keepdims=True))
    lsm = logits - lse
    out_ref[...] = lsm
    p = jnp.sum(jnp.where(valid, jnp.exp(lsm), 0.0) * r_ref[...], axis=1,
                keepdims=True)
    pred_ref[...] = jnp.broadcast_to(p, pred_ref.shape)


def _head(ha8, hb8, wh_W, wh_b, wp_W, wp_b, r):
    whw = jnp.zeros((2 * H, 64), jnp.float32).at[:, :HIDDEN].set(wh_W)
    whb = jnp.zeros((1, 64), jnp.float32).at[0, :HIDDEN].set(wh_b)
    wpw = jnp.zeros((64, 128), jnp.float32).at[:HIDDEN, :NUM_CLASSES].set(wp_W)
    wpb = jnp.zeros((1, 128), jnp.float32).at[0, :NUM_CLASSES].set(wp_b)
    rp = jnp.zeros((1, 128), jnp.float32).at[0, :NUM_CLASSES].set(r)
    out, pred = pl.pallas_call(
        _head_body,
        in_specs=[pl.BlockSpec((8, H), lambda: (0, 0)),
                  pl.BlockSpec((8, H), lambda: (0, 0)),
                  pl.BlockSpec((2 * H, 64), lambda: (0, 0)),
                  pl.BlockSpec((1, 64), lambda: (0, 0)),
                  pl.BlockSpec((64, 128), lambda: (0, 0)),
                  pl.BlockSpec((1, 128), lambda: (0, 0)),
                  pl.BlockSpec((1, 128), lambda: (0, 0))],
        out_specs=[pl.BlockSpec((8, 128), lambda: (0, 0)),
                   pl.BlockSpec((8, 128), lambda: (0, 0))],
        out_shape=[jax.ShapeDtypeStruct((8, 128), jnp.float32),
                   jax.ShapeDtypeStruct((8, 128), jnp.float32)],
    )(ha8, hb8, whw, whb, wpw, wpb, rp)
    return out[0:1, :NUM_CLASSES], pred[0:1, 0]


def _tree_setup(x_ids, parent, emb, W_iou, b_iou, W_f, b_f):
    ids = jnp.zeros((NP,), x_ids.dtype).at[:N].set(x_ids)
    parp = jnp.zeros((NP,), parent.dtype).at[:N].set(parent)
    x = jnp.take(emb, ids, axis=0)
    xw, xwf = _embed_mm(x, W_iou, b_iou, W_f, b_f)
    fp = jnp.take(xwf, parp, axis=0)
    gidx, dstl, cs = _build_edge_data(parent)
    c, h = _iter1(xw)
    return xw, fp, (gidx, dstl), cs, c, h


def kernel(x_ids_a, parent_a, x_ids_b, parent_b, emb, W_iou, U_iou, b_iou,
           W_f, U_f, b_f, wh_W, wh_b, wp_W, wp_b, r):
    xw_a, fp_a, ed_a, cs_a, c_a, h_a = _tree_setup(
        x_ids_a, parent_a, emb, W_iou, b_iou, W_f, b_f)
    xw_b, fp_b, ed_b, cs_b, c_b, h_b = _tree_setup(
        x_ids_b, parent_b, emb, W_iou, b_iou, W_f, b_f)
    csa = jnp.zeros((64,), jnp.int32).at[:CH + 1].set(cs_a)
    csb = jnp.zeros((64,), jnp.int32).at[:CH + 1].set(cs_b)
    zrows = jnp.zeros((UTR, 128), jnp.float32)
    for _ in range(K_ITERS - 1):
        hs_a = _scatter_add(h_a, ed_a, csa, zrows)
        hs_b = _scatter_add(h_b, ed_b, csb, zrows)
        iu_a, o_a, fcc_a = _step_a(xw_a, fp_a, h_a, hs_a, c_a, U_iou, U_f)
        iu_b, o_b, fcc_b = _step_a(xw_b, fp_b, h_b, hs_b, c_b, U_iou, U_f)
        fc_a = _scatter_add(fcc_a, ed_a, csa, zrows)
        fc_b = _scatter_add(fcc_b, ed_b, csb, zrows)
        c_a, h_a = _step_b(iu_a, o_a, fc_a)
        c_b, h_b = _step_b(iu_b, o_b, fc_b)
    return _head(h_a[0:8], h_b[0:8], wh_W, wh_b, wp_W, wp_b, r)


# speedup vs baseline: 1.3092x; 1.0026x over previous
import functools

import jax
import jax.numpy as jnp
from jax import lax
from jax.experimental import pallas as pl
from jax.experimental.pallas import tpu as pltpu
from jax.experimental.pallas import tpu_sc as plsc

N = 50000
X = 256
H = 256
K_ITERS = 6
HIDDEN = 50
NUM_CLASSES = 5

RB = 512
NP = ((N + RB - 1) // RB) * RB

T = 128
R = 2048
CH = 26
CPC = CH // 2
TRR = R // 16
TRASH = R


TE = T // 2
UTR = 2 * TRR
TRASHU = 2 * R


def _tree_pass(vals, gidx, dstl, out, cs_vm, cs_base, s, t0u,
               gi0, di0, rows0, gi1, di1, rows1, zbuf, acc, sem0, sem1, c):
    def chunk_body(j, carry):
        k = 2 * j + lax.bitwise_xor(c, lax.bitwise_and(j, 1))
        st = cs_vm[pl.ds(cs_base + k, 16)][0]
        en = cs_vm[pl.ds(cs_base + k + 1, 16)][0]
        cnt = en - st
        per = (cnt + 15) // 16
        s_t = st + s * per
        e_t = jnp.minimum(s_t + per, en)
        b = lax.bitwise_and(s_t, -4)
        nsteps = jnp.where(e_t > s_t, (e_t - b + TE - 1) // TE, 0)

        pltpu.sync_copy(zbuf, acc.at[pl.ds(t0u, UTR)])
        plsc.subcore_barrier()

        def load(i2, gi, di):
            qoff = pl.multiple_of(2 * b + i2 * T, 8)
            pltpu.sync_copy(gidx.at[pl.ds(qoff, T)], gi)
            pltpu.sync_copy(dstl.at[pl.ds(qoff, T)], di)

            def maskv(v, cc):
                q = qoff + v * 16 + lax.broadcasted_iota(jnp.int32, (16,), 0)
                e_idx = lax.shift_right_logical(q, 1)
                dv = di[pl.ds(v * 16, 16)]
                ok = (e_idx >= s_t) & (e_idx < e_t)
                di[pl.ds(v * 16, 16)] = jnp.where(ok, dv, TRASHU)
                return cc

            lax.fori_loop(0, T // 16, maskv, 0)

        def fire(gi, rows, sem):
            pltpu.make_async_copy(vals.at[gi], rows, sem).start()

        def drain(gi, di, rows, sem):
            pltpu.make_async_copy(vals.at[gi], rows, sem).wait()
            pltpu.async_copy(rows, acc.at[di], sem, add=True).wait()

        def step(i2, carry2):
            load(i2, gi0, di0)
            fire(gi0, rows0, sem0)
            drain(gi0, di0, rows0, sem0)
            return carry2

        lax.fori_loop(0, nsteps, step, 0)
        plsc.subcore_barrier()
        pltpu.sync_copy(acc.at[pl.ds(t0u, UTR)],
                        out.at[pl.ds(2 * k * R + t0u, UTR)])
        return carry

    lax.fori_loop(0, CPC, chunk_body, 0)


def _sc_scatter_body(vals, gidx, dstl, cs, zrows, out,
                     gi0, di0, rows0, gi1, di1, rows1,
                     zbuf, cs_vm, acc, sem0, sem1):
    c = lax.axis_index("c")
    s = lax.axis_index("s")
    pltpu.sync_copy(cs, cs_vm)
    pltpu.sync_copy(zrows, zbuf)
    t0u = s * UTR
    _tree_pass(vals, gidx, dstl, out, cs_vm, 0, s, t0u,
               gi0, di0, rows0, gi1, di1, rows1, zbuf, acc, sem0, sem1, c)


_sc_scatter = functools.partial(
    pl.kernel,
    mesh=plsc.VectorSubcoreMesh(core_axis_name="c", subcore_axis_name="s"),
    out_type=jax.ShapeDtypeStruct((2 * CH * R, 128), jnp.float32),
    scratch_types=[
        pltpu.VMEM((T,), jnp.int32),
        pltpu.VMEM((T,), jnp.int32),
        pltpu.VMEM((T, 128), jnp.float32),
        pltpu.VMEM((T,), jnp.int32),
        pltpu.VMEM((T,), jnp.int32),
        pltpu.VMEM((T, 128), jnp.float32),
        pltpu.VMEM((UTR, 128), jnp.float32),
        pltpu.VMEM((64,), jnp.int32),
        pltpu.VMEM_SHARED((2 * R + 16, 128), jnp.float32),
        pltpu.SemaphoreType.DMA,
        pltpu.SemaphoreType.DMA,
    ],
)(_sc_scatter_body)


def _scatter_add(arr, ed, cs2, zrows):
    v = jnp.reshape(arr, (2 * NP, 128))
    o = _sc_scatter(v, ed[0], ed[1], cs2, zrows)
    return jnp.reshape(o, (CH * R, H))


def _build_edge_data(parent):
    par = parent[1:N].astype(jnp.int32)
    order = jnp.argsort(par).astype(jnp.int32)
    ps = par[order]
    child = order + 1
    child2 = jnp.stack([2 * child, 2 * child + 1], axis=1).reshape(-1)
    du = 2 * (ps % R)
    dst2 = jnp.stack([du, du + 1], axis=1).reshape(-1)
    gidx = jnp.zeros((2 * NP,), jnp.int32).at[:2 * (N - 1)].set(child2)
    dstl = jnp.full((2 * NP,), TRASHU, jnp.int32).at[:2 * (N - 1)].set(dst2)
    bounds = jnp.arange(CH + 1, dtype=jnp.int32) * R
    cs = jnp.searchsorted(ps, bounds).astype(jnp.int32)
    return gidx, dstl, cs


def _embed_mm_body(x_ref, wiou_ref, biou_ref, wf_ref, bf_ref, xw_ref, xwf_ref):
    x = x_ref[...]
    xw_ref[...] = jnp.dot(x, wiou_ref[...],
                          preferred_element_type=jnp.float32) + biou_ref[...]
    xwf_ref[...] = jnp.dot(x, wf_ref[...],
                           preferred_element_type=jnp.float32) + bf_ref[...]


def _embed_mm(x, W_iou, b_iou, W_f, b_f):
    grid = (NP // RB,)
    return pl.pallas_call(
        _embed_mm_body,
        grid=grid,
        in_specs=[
            pl.BlockSpec((RB, X), lambda i: (i, 0)),
            pl.BlockSpec((X, 3 * H), lambda i: (0, 0)),
            pl.BlockSpec((1, 3 * H), lambda i: (0, 0)),
            pl.BlockSpec((X, H), lambda i: (0, 0)),
            pl.BlockSpec((1, H), lambda i: (0, 0)),
        ],
        out_specs=[
            pl.BlockSpec((RB, 3 * H), lambda i: (i, 0)),
            pl.BlockSpec((RB, H), lambda i: (i, 0)),
        ],
        out_shape=[
            jax.ShapeDtypeStruct((NP, 3 * H), jnp.float32),
            jax.ShapeDtypeStruct((NP, H), jnp.float32),
        ],
    )(x, W_iou, b_iou.reshape(1, -1), W_f, b_f.reshape(1, -1))


def _iter1_body(xw_ref, c_ref, h_ref):
    xw = xw_ref[...]
    i_g = jax.nn.sigmoid(xw[:, :H])
    o_g = jax.nn.sigmoid(xw[:, H:2 * H])
    u_g = jnp.tanh(xw[:, 2 * H:])
    c = i_g * u_g
    c_ref[...] = c
    h_ref[...] = o_g * jnp.tanh(c)


def _iter1(xw):
    grid = (NP // RB,)
    return pl.pallas_call(
        _iter1_body,
        grid=grid,
        in_specs=[pl.BlockSpec((RB, 3 * H), lambda i: (i, 0))],
        out_specs=[pl.BlockSpec((RB, H), lambda i: (i, 0)),
                   pl.BlockSpec((RB, H), lambda i: (i, 0))],
        out_shape=[jax.ShapeDtypeStruct((NP, H), jnp.float32),
                   jax.ShapeDtypeStruct((NP, H), jnp.float32)],
    )(xw)


def _step_a_body(xw_ref, fp_ref, h_ref, hs_ref, c_ref, uiou_ref, uf_ref,
                 iu_ref, o_ref, fcc_ref):
    iou = xw_ref[...] + jnp.dot(hs_ref[...], uiou_ref[...],
                                preferred_element_type=jnp.float32)
    i_g = jax.nn.sigmoid(iou[:, :H])
    o_g = jax.nn.sigmoid(iou[:, H:2 * H])
    u_g = jnp.tanh(iou[:, 2 * H:])
    iu_ref[...] = i_g * u_g
    o_ref[...] = o_g
    fe = jax.nn.sigmoid(fp_ref[...] + jnp.dot(h_ref[...], uf_ref[...],
                                              preferred_element_type=jnp.float32))
    fcc_ref[...] = fe * c_ref[...]


def _step_a(xw, fp, h, h_sum, c, U_iou, U_f):
    grid = (NP // RB,)
    return pl.pallas_call(
        _step_a_body,
        grid=grid,
        in_specs=[
            pl.BlockSpec((RB, 3 * H), lambda i: (i, 0)),
            pl.BlockSpec((RB, H), lambda i: (i, 0)),
            pl.BlockSpec((RB, H), lambda i: (i, 0)),
            pl.BlockSpec((RB, H), lambda i: (i, 0)),
            pl.BlockSpec((RB, H), lambda i: (i, 0)),
            pl.BlockSpec((H, 3 * H), lambda i: (0, 0)),
            pl.BlockSpec((H, H), lambda i: (0, 0)),
        ],
        out_specs=[pl.BlockSpec((RB, H), lambda i: (i, 0))] * 3,
        out_shape=[jax.ShapeDtypeStruct((NP, H), jnp.float32)] * 3,
    )(xw, fp, h, h_sum, c, U_iou, U_f)


def _step_b_body(iu_ref, o_ref, fc_ref, c_ref, h_ref):
    c = iu_ref[...] + fc_ref[...]
    c_ref[...] = c
    h_ref[...] = o_ref[...] * jnp.tanh(c)


def _step_b(iu, o, fc):
    grid = (NP // RB,)
    return pl.pallas_call(
        _step_b_body,
        grid=grid,
        in_specs=[pl.BlockSpec((RB, H), lambda i: (i, 0))] * 3,
        out_specs=[pl.BlockSpec((RB, H), lambda i: (i, 0))] * 2,
        out_shape=[jax.ShapeDtypeStruct((NP, H), jnp.float32)] * 2,
    )(iu, o, fc)


def _head_body(ha_ref, hb_ref, whw_ref, whb_ref, wpw_ref, wpb_ref, r_ref,
               out_ref, pred_ref):
    lvec = ha_ref[...]
    rvec = hb_ref[...]
    vec = jnp.concatenate([lvec * rvec, jnp.abs(lvec - rvec)], axis=1)
    hid = jax.nn.sigmoid(jnp.dot(vec, whw_ref[...],
                                 preferred_element_type=jnp.float32) + whb_ref[...])
    hcol = jax.lax.broadcasted_iota(jnp.int32, hid.shape, 1)
    hid = jnp.where(hcol < HIDDEN, hid, 0.0)
    logits = jnp.dot(hid, wpw_ref[...],
                     preferred_element_type=jnp.float32) + wpb_ref[...]
    col = jax.lax.broadcasted_iota(jnp.int32, logits.shape, 1)
    valid = col < NUM_CLASSES
    masked = jnp.where(valid, logits, -jnp.inf)
    m = jnp.max(masked, axis=1, keepdims=True)
    e = jnp.where(valid, jnp.exp(logits - m), 0.0)
    lse = m + jnp.log(jnp.sum(e, axis=1, keepdims=True))
    lsm = logits - lse
    out_ref[...] = lsm
    p = jnp.sum(jnp.where(valid, jnp.exp(lsm), 0.0) * r_ref[...], axis=1,
                keepdims=True)
    pred_ref[...] = jnp.broadcast_to(p, pred_ref.shape)


def _head(ha8, hb8, wh_W, wh_b, wp_W, wp_b, r):
    whw = jnp.zeros((2 * H, 64), jnp.float32).at[:, :HIDDEN].set(wh_W)
    whb = jnp.zeros((1, 64), jnp.float32).at[0, :HIDDEN].set(wh_b)
    wpw = jnp.zeros((64, 128), jnp.float32).at[:HIDDEN, :NUM_CLASSES].set(wp_W)
    wpb = jnp.zeros((1, 128), jnp.float32).at[0, :NUM_CLASSES].set(wp_b)
    rp = jnp.zeros((1, 128), jnp.float32).at[0, :NUM_CLASSES].set(r)
    out, pred = pl.pallas_call(
        _head_body,
        in_specs=[pl.BlockSpec((8, H), lambda: (0, 0)),
                  pl.BlockSpec((8, H), lambda: (0, 0)),
                  pl.BlockSpec((2 * H, 64), lambda: (0, 0)),
                  pl.BlockSpec((1, 64), lambda: (0, 0)),
                  pl.BlockSpec((64, 128), lambda: (0, 0)),
                  pl.BlockSpec((1, 128), lambda: (0, 0)),
                  pl.BlockSpec((1, 128), lambda: (0, 0))],
        out_specs=[pl.BlockSpec((8, 128), lambda: (0, 0)),
                   pl.BlockSpec((8, 128), lambda: (0, 0))],
        out_shape=[jax.ShapeDtypeStruct((8, 128), jnp.float32),
                   jax.ShapeDtypeStruct((8, 128), jnp.float32)],
    )(ha8, hb8, whw, whb, wpw, wpb, rp)
    return out[0:1, :NUM_CLASSES], pred[0:1, 0]


def _tree_setup(x_ids, parent, emb, W_iou, b_iou, W_f, b_f):
    ids = jnp.zeros((NP,), x_ids.dtype).at[:N].set(x_ids)
    parp = jnp.zeros((NP,), parent.dtype).at[:N].set(parent)
    x = jnp.take(emb, ids, axis=0)
    xw, xwf = _embed_mm(x, W_iou, b_iou, W_f, b_f)
    fp = jnp.take(xwf, parp, axis=0)
    gidx, dstl, cs = _build_edge_data(parent)
    c, h = _iter1(xw)
    return xw, fp, (gidx, dstl), cs, c, h


def kernel(x_ids_a, parent_a, x_ids_b, parent_b, emb, W_iou, U_iou, b_iou,
           W_f, U_f, b_f, wh_W, wh_b, wp_W, wp_b, r):
    xw_a, fp_a, ed_a, cs_a, c_a, h_a = _tree_setup(
        x_ids_a, parent_a, emb, W_iou, b_iou, W_f, b_f)
    xw_b, fp_b, ed_b, cs_b, c_b, h_b = _tree_setup(
        x_ids_b, parent_b, emb, W_iou, b_iou, W_f, b_f)
    csa = jnp.zeros((64,), jnp.int32).at[:CH + 1].set(cs_a)
    csb = jnp.zeros((64,), jnp.int32).at[:CH + 1].set(cs_b)
    zrows = jnp.zeros((UTR, 128), jnp.float32)
    for _ in range(K_ITERS - 1):
        hs_a = _scatter_add(h_a, ed_a, csa, zrows)
        hs_b = _scatter_add(h_b, ed_b, csb, zrows)
        iu_a, o_a, fcc_a = _step_a(xw_a, fp_a, h_a, hs_a, c_a, U_iou, U_f)
        iu_b, o_b, fcc_b = _step_a(xw_b, fp_b, h_b, hs_b, c_b, U_iou, U_f)
        fc_a = _scatter_add(fcc_a, ed_a, csa, zrows)
        fc_b = _scatter_add(fcc_b, ed_b, csb, zrows)
        c_a, h_a = _step_b(iu_a, o_a, fc_a)
        c_b, h_b = _step_b(iu_b, o_b, fc_b)
    return _head(h_a[0:8], h_b[0:8], wh_W, wh_b, wp_W, wp_b, r)


# speedup vs baseline: 1.3092x; 1.0001x over previous
import functools

import jax
import jax.numpy as jnp
from jax import lax
from jax.experimental import pallas as pl
from jax.experimental.pallas import tpu as pltpu
from jax.experimental.pallas import tpu_sc as plsc

N = 50000
X = 256
H = 256
K_ITERS = 6
HIDDEN = 50
NUM_CLASSES = 5

RB = 512
NP = ((N + RB - 1) // RB) * RB

T = 128
R = 2048
CH = 26
CPC = CH // 2
TRR = R // 16
TRASH = R


TE = T // 2
UTR = 2 * TRR
TRASHU = 2 * R


def _tree_pass(vals, gidx, dstl, out, cs_vm, cs_base, s, t0u,
               gi0, di0, rows0, zbuf, acc, sem0, c):
    def chunk_body(j, carry):
        k = 2 * j + lax.bitwise_xor(c, lax.bitwise_and(j, 1))
        st = cs_vm[pl.ds(cs_base + k, 16)][0]
        en = cs_vm[pl.ds(cs_base + k + 1, 16)][0]
        cnt = en - st
        per = (cnt + 15) // 16
        s_t = st + s * per
        e_t = jnp.minimum(s_t + per, en)
        b = lax.bitwise_and(s_t, -4)
        nsteps = jnp.where(e_t > s_t, (e_t - b + TE - 1) // TE, 0)

        pltpu.sync_copy(zbuf, acc.at[pl.ds(t0u, UTR)])
        plsc.subcore_barrier()

        def load(i2, gi, di):
            qoff = pl.multiple_of(2 * b + i2 * T, 8)
            pltpu.sync_copy(gidx.at[pl.ds(qoff, T)], gi)
            pltpu.sync_copy(dstl.at[pl.ds(qoff, T)], di)

            def maskv(v, cc):
                q = qoff + v * 16 + lax.broadcasted_iota(jnp.int32, (16,), 0)
                e_idx = lax.shift_right_logical(q, 1)
                dv = di[pl.ds(v * 16, 16)]
                ok = (e_idx >= s_t) & (e_idx < e_t)
                di[pl.ds(v * 16, 16)] = jnp.where(ok, dv, TRASHU)
                return cc

            lax.fori_loop(0, T // 16, maskv, 0)

        def fire(gi, rows, sem):
            pltpu.make_async_copy(vals.at[gi], rows, sem).start()

        def drain(gi, di, rows, sem):
            pltpu.make_async_copy(vals.at[gi], rows, sem).wait()
            pltpu.async_copy(rows, acc.at[di], sem, add=True).wait()

        def step(i2, carry2):
            load(i2, gi0, di0)
            fire(gi0, rows0, sem0)
            drain(gi0, di0, rows0, sem0)
            return carry2

        lax.fori_loop(0, nsteps, step, 0)
        plsc.subcore_barrier()
        pltpu.sync_copy(acc.at[pl.ds(t0u, UTR)],
                        out.at[pl.ds(2 * k * R + t0u, UTR)])
        return carry

    lax.fori_loop(0, CPC, chunk_body, 0)


def _sc_scatter_body(vals, gidx, dstl, cs, zrows, out,
                     gi0, di0, rows0, zbuf, cs_vm, acc, sem0):
    c = lax.axis_index("c")
    s = lax.axis_index("s")
    pltpu.sync_copy(cs, cs_vm)
    pltpu.sync_copy(zrows, zbuf)
    t0u = s * UTR
    _tree_pass(vals, gidx, dstl, out, cs_vm, 0, s, t0u,
               gi0, di0, rows0, zbuf, acc, sem0, c)


_sc_scatter = functools.partial(
    pl.kernel,
    mesh=plsc.VectorSubcoreMesh(core_axis_name="c", subcore_axis_name="s"),
    out_type=jax.ShapeDtypeStruct((2 * CH * R, 128), jnp.float32),
    scratch_types=[
        pltpu.VMEM((T,), jnp.int32),
        pltpu.VMEM((T,), jnp.int32),
        pltpu.VMEM((T, 128), jnp.float32),
        pltpu.VMEM((UTR, 128), jnp.float32),
        pltpu.VMEM((64,), jnp.int32),
        pltpu.VMEM_SHARED((2 * R + 16, 128), jnp.float32),
        pltpu.SemaphoreType.DMA,
    ],
)(_sc_scatter_body)


def _scatter_add(arr, ed, cs2, zrows):
    v = jnp.reshape(arr, (2 * NP, 128))
    o = _sc_scatter(v, ed[0], ed[1], cs2, zrows)
    return jnp.reshape(o, (CH * R, H))


def _build_edge_data(parent):
    par = parent[1:N].astype(jnp.int32)
    order = jnp.argsort(par).astype(jnp.int32)
    ps = par[order]
    child = order + 1
    child2 = jnp.stack([2 * child, 2 * child + 1], axis=1).reshape(-1)
    du = 2 * (ps % R)
    dst2 = jnp.stack([du, du + 1], axis=1).reshape(-1)
    gidx = jnp.zeros((2 * NP,), jnp.int32).at[:2 * (N - 1)].set(child2)
    dstl = jnp.full((2 * NP,), TRASHU, jnp.int32).at[:2 * (N - 1)].set(dst2)
    bounds = jnp.arange(CH + 1, dtype=jnp.int32) * R
    cs = jnp.searchsorted(ps, bounds).astype(jnp.int32)
    return gidx, dstl, cs


def _embed_mm_body(x_ref, wiou_ref, biou_ref, wf_ref, bf_ref, xw_ref, xwf_ref):
    x = x_ref[...]
    xw_ref[...] = jnp.dot(x, wiou_ref[...],
                          preferred_element_type=jnp.float32) + biou_ref[...]
    xwf_ref[...] = jnp.dot(x, wf_ref[...],
                           preferred_element_type=jnp.float32) + bf_ref[...]


def _embed_mm(x, W_iou, b_iou, W_f, b_f):
    grid = (NP // RB,)
    return pl.pallas_call(
        _embed_mm_body,
        grid=grid,
        in_specs=[
            pl.BlockSpec((RB, X), lambda i: (i, 0)),
            pl.BlockSpec((X, 3 * H), lambda i: (0, 0)),
            pl.BlockSpec((1, 3 * H), lambda i: (0, 0)),
            pl.BlockSpec((X, H), lambda i: (0, 0)),
            pl.BlockSpec((1, H), lambda i: (0, 0)),
        ],
        out_specs=[
            pl.BlockSpec((RB, 3 * H), lambda i: (i, 0)),
            pl.BlockSpec((RB, H), lambda i: (i, 0)),
        ],
        out_shape=[
            jax.ShapeDtypeStruct((NP, 3 * H), jnp.float32),
            jax.ShapeDtypeStruct((NP, H), jnp.float32),
        ],
    )(x, W_iou, b_iou.reshape(1, -1), W_f, b_f.reshape(1, -1))


def _iter1_body(xw_ref, c_ref, h_ref):
    xw = xw_ref[...]
    i_g = jax.nn.sigmoid(xw[:, :H])
    o_g = jax.nn.sigmoid(xw[:, H:2 * H])
    u_g = jnp.tanh(xw[:, 2 * H:])
    c = i_g * u_g
    c_ref[...] = c
    h_ref[...] = o_g * jnp.tanh(c)


def _iter1(xw):
    grid = (NP // RB,)
    return pl.pallas_call(
        _iter1_body,
        grid=grid,
        in_specs=[pl.BlockSpec((RB, 3 * H), lambda i: (i, 0))],
        out_specs=[pl.BlockSpec((RB, H), lambda i: (i, 0)),
                   pl.BlockSpec((RB, H), lambda i: (i, 0))],
        out_shape=[jax.ShapeDtypeStruct((NP, H), jnp.float32),
                   jax.ShapeDtypeStruct((NP, H), jnp.float32)],
    )(xw)


def _step_a_body(xw_ref, fp_ref, h_ref, hs_ref, c_ref, uiou_ref, uf_ref,
                 iu_ref, o_ref, fcc_ref):
    iou = xw_ref[...] + jnp.dot(hs_ref[...], uiou_ref[...],
                                preferred_element_type=jnp.float32)
    i_g = jax.nn.sigmoid(iou[:, :H])
    o_g = jax.nn.sigmoid(iou[:, H:2 * H])
    u_g = jnp.tanh(iou[:, 2 * H:])
    iu_ref[...] = i_g * u_g
    o_ref[...] = o_g
    fe = jax.nn.sigmoid(fp_ref[...] + jnp.dot(h_ref[...], uf_ref[...],
                                              preferred_element_type=jnp.float32))
    fcc_ref[...] = fe * c_ref[...]


def _step_a(xw, fp, h, h_sum, c, U_iou, U_f):
    grid = (NP // RB,)
    return pl.pallas_call(
        _step_a_body,
        grid=grid,
        in_specs=[
            pl.BlockSpec((RB, 3 * H), lambda i: (i, 0)),
            pl.BlockSpec((RB, H), lambda i: (i, 0)),
            pl.BlockSpec((RB, H), lambda i: (i, 0)),
            pl.BlockSpec((RB, H), lambda i: (i, 0)),
            pl.BlockSpec((RB, H), lambda i: (i, 0)),
            pl.BlockSpec((H, 3 * H), lambda i: (0, 0)),
            pl.BlockSpec((H, H), lambda i: (0, 0)),
        ],
        out_specs=[pl.BlockSpec((RB, H), lambda i: (i, 0))] * 3,
        out_shape=[jax.ShapeDtypeStruct((NP, H), jnp.float32)] * 3,
    )(xw, fp, h, h_sum, c, U_iou, U_f)


def _step_b_body(iu_ref, o_ref, fc_ref, c_ref, h_ref):
    c = iu_ref[...] + fc_ref[...]
    c_ref[...] = c
    h_ref[...] = o_ref[...] * jnp.tanh(c)


def _step_b(iu, o, fc):
    grid = (NP // RB,)
    return pl.pallas_call(
        _step_b_body,
        grid=grid,
        in_specs=[pl.BlockSpec((RB, H), lambda i: (i, 0))] * 3,
        out_specs=[pl.BlockSpec((RB, H), lambda i: (i, 0))] * 2,
        out_shape=[jax.ShapeDtypeStruct((NP, H), jnp.float32)] * 2,
    )(iu, o, fc)


def _head_body(ha_ref, hb_ref, whw_ref, whb_ref, wpw_ref, wpb_ref, r_ref,
               out_ref, pred_ref):
    lvec = ha_ref[...]
    rvec = hb_ref[...]
    vec = jnp.concatenate([lvec * rvec, jnp.abs(lvec - rvec)], axis=1)
    hid = jax.nn.sigmoid(jnp.dot(vec, whw_ref[...],
                                 preferred_element_type=jnp.float32) + whb_ref[...])
    hcol = jax.lax.broadcasted_iota(jnp.int32, hid.shape, 1)
    hid = jnp.where(hcol < HIDDEN, hid, 0.0)
    logits = jnp.dot(hid, wpw_ref[...],
                     preferred_element_type=jnp.float32) + wpb_ref[...]
    col = jax.lax.broadcasted_iota(jnp.int32, logits.shape, 1)
    valid = col < NUM_CLASSES
    masked = jnp.where(valid, logits, -jnp.inf)
    m = jnp.max(masked, axis=1, keepdims=True)
    e = jnp.where(valid, jnp.exp(logits - m), 0.0)
    lse = m + jnp.log(jnp.sum(e, axis=1, keepdims=True))
    lsm = logits - lse
    out_ref[...] = lsm
    p = jnp.sum(jnp.where(valid, jnp.exp(lsm), 0.0) * r_ref[...], axis=1,
                keepdims=True)
    pred_ref[...] = jnp.broadcast_to(p, pred_ref.shape)


def _head(ha8, hb8, wh_W, wh_b, wp_W, wp_b, r):
    whw = jnp.zeros((2 * H, 64), jnp.float32).at[:, :HIDDEN].set(wh_W)
    whb = jnp.zeros((1, 64), jnp.float32).at[0, :HIDDEN].set(wh_b)
    wpw = jnp.zeros((64, 128), jnp.float32).at[:HIDDEN, :NUM_CLASSES].set(wp_W)
    wpb = jnp.zeros((1, 128), jnp.float32).at[0, :NUM_CLASSES].set(wp_b)
    rp = jnp.zeros((1, 128), jnp.float32).at[0, :NUM_CLASSES].set(r)
    out, pred = pl.pallas_call(
        _head_body,
        in_specs=[pl.BlockSpec((8, H), lambda: (0, 0)),
                  pl.BlockSpec((8, H), lambda: (0, 0)),
                  pl.BlockSpec((2 * H, 64), lambda: (0, 0)),
                  pl.BlockSpec((1, 64), lambda: (0, 0)),
                  pl.BlockSpec((64, 128), lambda: (0, 0)),
                  pl.BlockSpec((1, 128), lambda: (0, 0)),
                  pl.BlockSpec((1, 128), lambda: (0, 0))],
        out_specs=[pl.BlockSpec((8, 128), lambda: (0, 0)),
                   pl.BlockSpec((8, 128), lambda: (0, 0))],
        out_shape=[jax.ShapeDtypeStruct((8, 128), jnp.float32),
                   jax.ShapeDtypeStruct((8, 128), jnp.float32)],
    )(ha8, hb8, whw, whb, wpw, wpb, rp)
    return out[0:1, :NUM_CLASSES], pred[0:1, 0]


def _tree_setup(x_ids, parent, emb, W_iou, b_iou, W_f, b_f):
    ids = jnp.zeros((NP,), x_ids.dtype).at[:N].set(x_ids)
    parp = jnp.zeros((NP,), parent.dtype).at[:N].set(parent)
    x = jnp.take(emb, ids, axis=0)
    xw, xwf = _embed_mm(x, W_iou, b_iou, W_f, b_f)
    fp = jnp.take(xwf, parp, axis=0)
    gidx, dstl, cs = _build_edge_data(parent)
    c, h = _iter1(xw)
    return xw, fp, (gidx, dstl), cs, c, h


def kernel(x_ids_a, parent_a, x_ids_b, parent_b, emb, W_iou, U_iou, b_iou,
           W_f, U_f, b_f, wh_W, wh_b, wp_W, wp_b, r):
    xw_a, fp_a, ed_a, cs_a, c_a, h_a = _tree_setup(
        x_ids_a, parent_a, emb, W_iou, b_iou, W_f, b_f)
    xw_b, fp_b, ed_b, cs_b, c_b, h_b = _tree_setup(
        x_ids_b, parent_b, emb, W_iou, b_iou, W_f, b_f)
    csa = jnp.zeros((64,), jnp.int32).at[:CH + 1].set(cs_a)
    csb = jnp.zeros((64,), jnp.int32).at[:CH + 1].set(cs_b)
    zrows = jnp.zeros((UTR, 128), jnp.float32)
    for _ in range(K_ITERS - 1):
        hs_a = _scatter_add(h_a, ed_a, csa, zrows)
        hs_b = _scatter_add(h_b, ed_b, csb, zrows)
        iu_a, o_a, fcc_a = _step_a(xw_a, fp_a, h_a, hs_a, c_a, U_iou, U_f)
        iu_b, o_b, fcc_b = _step_a(xw_b, fp_b, h_b, hs_b, c_b, U_iou, U_f)
        fc_a = _scatter_add(fcc_a, ed_a, csa, zrows)
        fc_b = _scatter_add(fcc_b, ed_b, csb, zrows)
        c_a, h_a = _step_b(iu_a, o_a, fc_a)
        c_b, h_b = _step_b(iu_b, o_b, fc_b)
    return _head(h_a[0:8], h_b[0:8], wh_W, wh_b, wp_W, wp_b, r)
